# Initial kernel scaffold; baseline (speedup 1.0000x reference)
import functools
import math

import jax
import jax.numpy as jnp
from jax import lax
from jax.experimental import pallas as pl
from jax.experimental.pallas import tpu as pltpu
from jax.experimental.pallas import tpu_sc as plsc

M = 1048576
N = 262144
DIST_TH2 = 0.05 * 0.05
DOT_TH = math.cos(20.0 * math.pi / 180.0)
DOT_TH2 = DOT_TH * DOT_TH
SIGMA = 0.6
INV_2SIG2 = 1.0 / (2.0 * SIGMA * SIGMA)

NC = 2
NS = 16
NW = NC * NS
NPW = N // NW


def _iota16():
  return lax.iota(jnp.int32, 16)


def _col(ref, rows, c):
  return plsc.load_gather(ref, [rows, jnp.full((16,), c, jnp.int32)])


def _sc_find_fuse(mp, mn, mc, mconf, fp, fn, fc, idx):
  mesh = plsc.VectorSubcoreMesh(core_axis_name="c", subcore_axis_name="s")

  @functools.partial(
      pl.kernel,
      out_type=(
          jax.ShapeDtypeStruct((NW, 16), jnp.int32),
          jax.ShapeDtypeStruct((N,), jnp.int32),
          jax.ShapeDtypeStruct((N, 10), jnp.float32),
      ),
      mesh=mesh,
      scratch_types=[
          pltpu.VMEM((NPW,), jnp.int32),
          pltpu.VMEM((NPW, 3), jnp.float32),
          pltpu.VMEM((NPW, 3), jnp.float32),
          pltpu.VMEM((NPW, 3), jnp.float32),
          pltpu.VMEM((NPW, 3), jnp.float32),
          pltpu.VMEM((NPW + 16,), jnp.int32),
          pltpu.VMEM((NPW + 16,), jnp.int32),
          pltpu.VMEM((16,), jnp.int32),
          pltpu.VMEM((16,), jnp.int32),
          pltpu.VMEM((16,), jnp.int32),
          pltpu.VMEM((16, 3), jnp.float32),
          pltpu.VMEM((16, 3), jnp.float32),
          pltpu.VMEM((16, 3), jnp.float32),
          pltpu.VMEM((16, 3), jnp.float32),
          pltpu.VMEM((16, 3), jnp.float32),
          pltpu.VMEM((16, 3), jnp.float32),
          pltpu.VMEM((16,), jnp.float32),
          pltpu.VMEM((16, 10), jnp.float32),
          pltpu.SemaphoreType.DMA,
      ],
  )
  def k(mp_h, mn_h, mc_h, mconf_h, fp_h, fn_h, fc_h, idx_h,
        counts_h, eidx_h, erow_h,
        idx_v, fp_v, fn_v, mp_v, mn_v, gi_list, idx_list,
        cnt_v, gi16, idx16, fp16, fn16, fc16, mp16, mn16, mc16, conf16,
        row16, sem):
    wid = lax.axis_index("s") * NC + lax.axis_index("c")
    base = wid * NPW
    iot = _iota16()

    d1 = pltpu.async_copy(idx_h.at[pl.ds(base, NPW)], idx_v, sem)
    d2 = pltpu.async_copy(fp_h.at[pl.ds(base, NPW)], fp_v, sem)
    d3 = pltpu.async_copy(fn_h.at[pl.ds(base, NPW)], fn_v, sem)
    d1.wait()
    d4 = pltpu.async_copy(mp_h.at[idx_v], mp_v, sem)
    d5 = pltpu.async_copy(mn_h.at[idx_v], mn_v, sem)
    d2.wait()
    d3.wait()
    d4.wait()
    d5.wait()

    def group(g, ptr):
      rows = g * 16 + iot
      fpx = _col(fp_v, rows, 0)
      fpy = _col(fp_v, rows, 1)
      fpz = _col(fp_v, rows, 2)
      mpx = _col(mp_v, rows, 0)
      mpy = _col(mp_v, rows, 1)
      mpz = _col(mp_v, rows, 2)
      dx = fpx - mpx
      dy = fpy - mpy
      dz = fpz - mpz
      d2_ = dx * dx + dy * dy + dz * dz
      fnx = _col(fn_v, rows, 0)
      fny = _col(fn_v, rows, 1)
      fnz = _col(fn_v, rows, 2)
      mnx = _col(mn_v, rows, 0)
      mny = _col(mn_v, rows, 1)
      mnz = _col(mn_v, rows, 2)
      dot = mnx * fnx + mny * fny + mnz * fnz
      nm2 = mnx * mnx + mny * mny + mnz * mnz
      nf2 = fnx * fnx + fny * fny + fnz * fnz
      valid = ((d2_ < DIST_TH2) & (dot > 0.0)
               & (dot * dot > DOT_TH2 * nm2 * nf2))
      cnt = jnp.max(plsc.all_reduce_population_count(valid))

      @pl.when(cnt > 0)
      def _():
        gi_vals = base + rows
        idx16v = idx_v[pl.ds(g * 16, 16)]
        plsc.store_compressed(gi_list.at[pl.ds(ptr, 16)], gi_vals, mask=valid)
        plsc.store_compressed(idx_list.at[pl.ds(ptr, 16)], idx16v, mask=valid)

      return ptr + cnt

    kcount = lax.fori_loop(0, NPW // 16, group, jnp.int32(0))

    cnt_v[...] = jnp.full((16,), kcount, jnp.int32)
    pltpu.sync_copy(cnt_v, counts_h.at[wid])

    zeros16 = jnp.zeros((16,), jnp.int32)
    gi_list[pl.ds(kcount, 16)] = zeros16
    idx_list[pl.ds(kcount, 16)] = zeros16

    def fix(t, _):
      e = t * 16
      gi16[...] = gi_list[pl.ds(e, 16)]
      idx16[...] = idx_list[pl.ds(e, 16)]
      g1 = pltpu.async_copy(fp_h.at[gi16], fp16, sem)
      g2 = pltpu.async_copy(fn_h.at[gi16], fn16, sem)
      g3 = pltpu.async_copy(fc_h.at[gi16], fc16, sem)
      g4 = pltpu.async_copy(mp_h.at[idx16], mp16, sem)
      g5 = pltpu.async_copy(mn_h.at[idx16], mn16, sem)
      g6 = pltpu.async_copy(mc_h.at[idx16], mc16, sem)
      g7 = pltpu.async_copy(mconf_h.at[idx16], conf16, sem)
      g1.wait()
      g2.wait()
      g3.wait()
      g4.wait()
      g5.wait()
      g6.wait()
      g7.wait()

      fpx = _col(fp16, iot, 0)
      fpy = _col(fp16, iot, 1)
      fpz = _col(fp16, iot, 2)
      r2 = fpx * fpx + fpy * fpy + fpz * fpz
      dc2 = (fpx * fpx + fpy * fpy) / (r2 + 1e-30)
      alpha = jnp.exp(-dc2 * INV_2SIG2)
      w = conf16[...]
      inv_den = 1.0 / (w + alpha)
      out_cols = []
      for c in range(3):
        out_cols.append((w * _col(mp16, iot, c)
                         + alpha * _col(fp16, iot, c)) * inv_den)
      for c in range(3):
        out_cols.append((w * _col(mn16, iot, c)
                         + alpha * _col(fn16, iot, c)) * inv_den)
      for c in range(3):
        out_cols.append((w * _col(mc16, iot, c)
                         + alpha * _col(fc16, iot, c)) * inv_den)
      out_cols.append(w + alpha)
      for c in range(10):
        plsc.store_scatter(row16, [iot, jnp.full((16,), c, jnp.int32)],
                           out_cols[c])
      pltpu.sync_copy(row16, erow_h.at[pl.ds(base + e, 16)])
      pltpu.sync_copy(idx16, eidx_h.at[pl.ds(base + e, 16)])
      return 0

    lax.fori_loop(0, (kcount + 15) // 16, fix, 0)

  return k(mp, mn, mc, mconf, fp, fn, fc, idx)


BLK = 4096


def _pack_body(p_ref, n_ref, c_ref, cf_ref, o_ref):
  o_ref[:, 0:3] = p_ref[...]
  o_ref[:, 3:6] = n_ref[...]
  o_ref[:, 6:9] = c_ref[...]
  o_ref[:, 9:10] = cf_ref[...]


def _tc_pack(mp, mn, mc, mconf):
  return pl.pallas_call(
      _pack_body,
      grid=(M // BLK,),
      in_specs=[
          pl.BlockSpec((BLK, 3), lambda i: (i, 0)),
          pl.BlockSpec((BLK, 3), lambda i: (i, 0)),
          pl.BlockSpec((BLK, 3), lambda i: (i, 0)),
          pl.BlockSpec((BLK, 1), lambda i: (i, 0)),
      ],
      out_specs=pl.BlockSpec((BLK, 10), lambda i: (i, 0)),
      out_shape=jax.ShapeDtypeStruct((M, 10), jnp.float32),
  )(mp, mn, mc, mconf.reshape(M, 1))


def _apply_body(counts_ref, eidx_ref, erow_ref, out_in_ref, out_ref,
                idx_sm, row_v):
  del out_in_ref
  for w in range(NW):
    cw = counts_ref[w, 0]

    def body(e, _, w=w):
      j = w * NPW + e
      pltpu.sync_copy(eidx_ref.at[pl.ds(j, 1)], idx_sm)
      m = idx_sm[0]
      pltpu.sync_copy(erow_ref.at[pl.ds(j, 1)], row_v)
      pltpu.sync_copy(row_v, out_ref.at[pl.ds(m, 1)])
      return 0

    lax.fori_loop(0, cw, body, 0)


def _tc_apply(counts, eidx, erow, out0):
  return pl.pallas_call(
      _apply_body,
      in_specs=[
          pl.BlockSpec(memory_space=pltpu.SMEM),
          pl.BlockSpec(memory_space=pltpu.ANY),
          pl.BlockSpec(memory_space=pltpu.ANY),
          pl.BlockSpec(memory_space=pltpu.ANY),
      ],
      out_specs=pl.BlockSpec(memory_space=pltpu.ANY),
      out_shape=jax.ShapeDtypeStruct((M, 10), jnp.float32),
      scratch_shapes=[
          pltpu.SMEM((1,), jnp.int32),
          pltpu.VMEM((1, 10), jnp.float32),
      ],
      input_output_aliases={3: 0},
  )(counts, eidx, erow, out0)


def kernel(map_points, map_normals, map_colors, map_confidence,
           frame_points, frame_normals, frame_colors, idx):
  counts, eidx, erow = _sc_find_fuse(
      map_points, map_normals, map_colors, map_confidence,
      frame_points, frame_normals, frame_colors, idx)
  out0 = _tc_pack(map_points, map_normals, map_colors, map_confidence)
  return _tc_apply(counts, eidx, erow, out0)


# speedup vs baseline: 1.8147x; 1.8147x over previous
import functools
import math

import jax
import jax.numpy as jnp
from jax import lax
from jax.experimental import pallas as pl
from jax.experimental.pallas import tpu as pltpu
from jax.experimental.pallas import tpu_sc as plsc

M = 1048576
N = 262144
DIST_TH2 = 0.05 * 0.05
DOT_TH = math.cos(20.0 * math.pi / 180.0)
DOT_TH2 = DOT_TH * DOT_TH
SIGMA = 0.6
INV_2SIG2 = 1.0 / (2.0 * SIGMA * SIGMA)

NC = 2
NS = 16
NW = NC * NS
NPW = N // NW
CH = NPW // 4
NGC = CH // 16
RPW = M // NW
CHP = 2048
NGP = CHP // 16

_SC_PARAMS = pltpu.CompilerParams(
    needs_layout_passes=False, use_tc_tiling_on_sc=False)


def _col(ref, rows, c):
  return plsc.load_gather(ref, [rows, jnp.full((16,), c, jnp.int32)])


def _sc_find_fuse(mp, mn, mc, mconf, fp, fn, fc, idx):
  mesh = plsc.VectorSubcoreMesh(core_axis_name="c", subcore_axis_name="s")

  @functools.partial(
      pl.kernel,
      out_type=(
          jax.ShapeDtypeStruct((NW, 16), jnp.int32),
          jax.ShapeDtypeStruct((N,), jnp.int32),
          jax.ShapeDtypeStruct((N, 10), jnp.float32),
      ),
      mesh=mesh,
      compiler_params=_SC_PARAMS,
      scratch_types=[
          pltpu.VMEM((CH,), jnp.int32),
          pltpu.VMEM((CH, 3), jnp.float32),
          pltpu.VMEM((CH, 3), jnp.float32),
          pltpu.VMEM((CH, 3), jnp.float32),
          pltpu.VMEM((CH, 3), jnp.float32),
          pltpu.VMEM((NPW + 16,), jnp.int32),
          pltpu.VMEM((NPW + 16,), jnp.int32),
          pltpu.VMEM((16,), jnp.int32),
          pltpu.VMEM((16,), jnp.int32),
          pltpu.VMEM((16,), jnp.int32),
          pltpu.VMEM((16, 3), jnp.float32),
          pltpu.VMEM((16, 3), jnp.float32),
          pltpu.VMEM((16, 3), jnp.float32),
          pltpu.VMEM((16, 3), jnp.float32),
          pltpu.VMEM((16, 3), jnp.float32),
          pltpu.VMEM((16, 3), jnp.float32),
          pltpu.VMEM((16,), jnp.float32),
          pltpu.VMEM((16, 10), jnp.float32),
          pltpu.SemaphoreType.DMA,
      ],
  )
  def k(mp_h, mn_h, mc_h, mconf_h, fp_h, fn_h, fc_h, idx_h,
        counts_h, eidx_h, erow_h,
        idx_v, fp_v, fn_v, mp_v, mn_v, gi_list, idx_list,
        cnt_v, gi16, idx16, fp16, fn16, fc16, mp16, mn16, mc16, conf16,
        row16, sem):
    wid = lax.axis_index("s") * NC + lax.axis_index("c")
    base = wid * NPW
    iot = lax.iota(jnp.int32, 16)

    kcount = jnp.int32(0)
    for half in range(NPW // CH):
      cbase = base + half * CH

      d1 = pltpu.async_copy(idx_h.at[pl.ds(cbase, CH)], idx_v, sem)
      d2 = pltpu.async_copy(fp_h.at[pl.ds(cbase, CH)], fp_v, sem)
      d3 = pltpu.async_copy(fn_h.at[pl.ds(cbase, CH)], fn_v, sem)
      d1.wait()
      g1 = pltpu.async_copy(mp_h.at[idx_v], mp_v, sem)
      g2 = pltpu.async_copy(mn_h.at[idx_v], mn_v, sem)
      d2.wait()
      d3.wait()
      g1.wait()
      g2.wait()

      def group(g, ptr, cbase=cbase):
        sl = pl.ds(g * 16, 16)
        rows = g * 16 + iot
        fpx = _col(fp_v, rows, 0)
        fpy = _col(fp_v, rows, 1)
        fpz = _col(fp_v, rows, 2)
        mpx = _col(mp_v, rows, 0)
        mpy = _col(mp_v, rows, 1)
        mpz = _col(mp_v, rows, 2)
        dx = fpx - mpx
        dy = fpy - mpy
        dz = fpz - mpz
        d2_ = dx * dx + dy * dy + dz * dz
        fnx = _col(fn_v, rows, 0)
        fny = _col(fn_v, rows, 1)
        fnz = _col(fn_v, rows, 2)
        mnx = _col(mn_v, rows, 0)
        mny = _col(mn_v, rows, 1)
        mnz = _col(mn_v, rows, 2)
        dot = mnx * fnx + mny * fny + mnz * fnz
        nm2 = mnx * mnx + mny * mny + mnz * mnz
        nf2 = fnx * fnx + fny * fny + fnz * fnz
        valid = ((d2_ < DIST_TH2) & (dot > 0.0)
                 & (dot * dot > DOT_TH2 * nm2 * nf2))
        cnt = jnp.max(plsc.all_reduce_population_count(valid))

        @pl.when(cnt > 0)
        def _():
          gi_vals = cbase + rows
          idx16v = idx_v[sl]
          plsc.store_compressed(gi_list.at[pl.ds(ptr, 16)], gi_vals,
                                mask=valid)
          plsc.store_compressed(idx_list.at[pl.ds(ptr, 16)], idx16v,
                                mask=valid)

        return ptr + cnt

      kcount = lax.fori_loop(0, NGC, group, kcount)

    nfix = (kcount + 15) // 16
    cnt_v[...] = jnp.full((16,), nfix * 16, jnp.int32)
    pltpu.sync_copy(cnt_v, counts_h.at[wid])

    zeros16 = jnp.zeros((16,), jnp.int32)
    gi_list[pl.ds(kcount, 16)] = zeros16
    idx_list[pl.ds(kcount, 16)] = zeros16

    def gather16(src_h, iref, dst):
      return pltpu.async_copy(src_h.at[iref], dst, sem)

    def fix(t, _):
      e = t * 16
      gi16[...] = gi_list[pl.ds(e, 16)]
      idx16[...] = idx_list[pl.ds(e, 16)]
      h1 = gather16(fp_h, gi16, fp16)
      h2 = gather16(fn_h, gi16, fn16)
      h3 = gather16(fc_h, gi16, fc16)
      h4 = gather16(mp_h, idx16, mp16)
      h5 = gather16(mn_h, idx16, mn16)
      h6 = gather16(mc_h, idx16, mc16)
      h7 = gather16(mconf_h, idx16, conf16)
      h1.wait()
      h2.wait()
      h3.wait()
      h4.wait()
      h5.wait()
      h6.wait()
      h7.wait()

      real = (e + iot) < kcount
      fpx = _col(fp16, iot, 0)
      fpy = _col(fp16, iot, 1)
      fpz = _col(fp16, iot, 2)
      r2 = fpx * fpx + fpy * fpy + fpz * fpz
      dc2 = (fpx * fpx + fpy * fpy) / (r2 + 1e-30)
      alpha = jnp.exp(-dc2 * INV_2SIG2)
      w = conf16[...]
      inv_den = 1.0 / (w + alpha)
      wd = jnp.where(real, w * inv_den, 1.0)
      ad = jnp.where(real, alpha * inv_den, 0.0)

      c10 = jnp.full((16,), 9, jnp.int32)
      plsc.store_scatter(row16, [iot, c10], jnp.where(real, w + alpha, w))
      for c in range(3):
        cc = jnp.full((16,), c, jnp.int32)
        plsc.store_scatter(
            row16, [iot, cc],
            wd * _col(mp16, iot, c) + ad * _col(fp16, iot, c))
      for c in range(3):
        cc = jnp.full((16,), 3 + c, jnp.int32)
        plsc.store_scatter(
            row16, [iot, cc],
            wd * _col(mn16, iot, c) + ad * _col(fn16, iot, c))
      for c in range(3):
        cc = jnp.full((16,), 6 + c, jnp.int32)
        plsc.store_scatter(
            row16, [iot, cc],
            wd * _col(mc16, iot, c) + ad * _col(fc16, iot, c))

      pltpu.sync_copy(row16, erow_h.at[pl.ds(base + e, 16)])
      pltpu.sync_copy(idx16, eidx_h.at[pl.ds(base + e, 16)])
      return 0

    lax.fori_loop(0, nfix, fix, 0)

  return k(mp, mn, mc, mconf, fp, fn, fc, idx)


def _sc_pack_apply(mp, mn, mc, mconf, counts, eidx, erow):
  mesh = plsc.VectorSubcoreMesh(core_axis_name="c", subcore_axis_name="s")

  @functools.partial(
      pl.kernel,
      out_type=jax.ShapeDtypeStruct((M, 10), jnp.float32),
      mesh=mesh,
      compiler_params=_SC_PARAMS,
      scratch_types=[
          pltpu.VMEM((CHP, 3), jnp.float32),
          pltpu.VMEM((CHP, 3), jnp.float32),
          pltpu.VMEM((CHP, 3), jnp.float32),
          pltpu.VMEM((CHP,), jnp.float32),
          pltpu.VMEM((CHP, 10), jnp.float32),
          pltpu.VMEM((NW, 16), jnp.int32),
          pltpu.VMEM((16,), jnp.int32),
          pltpu.VMEM((16, 10), jnp.float32),
          pltpu.SemaphoreType.DMA,
      ],
  )
  def k(mp_h, mn_h, mc_h, mconf_h, counts_h, eidx_h, erow_h, out_h,
        mp_pv, mn_pv, mc_pv, cf_pv, ob_v, counts_v, tgt16, rb_v, sem):
    wid = lax.axis_index("s") * NC + lax.axis_index("c")
    rbase = wid * RPW
    iot = lax.iota(jnp.int32, 16)
    dcnt = pltpu.async_copy(counts_h, counts_v, sem)

    for chunk in range(RPW // CHP):
      r0 = rbase + chunk * CHP
      d1 = pltpu.async_copy(mp_h.at[pl.ds(r0, CHP)], mp_pv, sem)
      d2 = pltpu.async_copy(mn_h.at[pl.ds(r0, CHP)], mn_pv, sem)
      d3 = pltpu.async_copy(mc_h.at[pl.ds(r0, CHP)], mc_pv, sem)
      d4 = pltpu.async_copy(mconf_h.at[pl.ds(r0, CHP)], cf_pv, sem)
      d1.wait()
      d2.wait()
      d3.wait()
      d4.wait()

      def pgroup(g, _):
        rows = g * 16 + iot
        for off, src in ((0, mp_pv), (3, mn_pv), (6, mc_pv)):
          for c in range(3):
            cc = jnp.full((16,), off + c, jnp.int32)
            plsc.store_scatter(ob_v, [rows, cc], _col(src, rows, c))
        c9 = jnp.full((16,), 9, jnp.int32)
        plsc.store_scatter(ob_v, [rows, c9], cf_pv[pl.ds(g * 16, 16)])
        return 0

      lax.fori_loop(0, NGP, pgroup, 0)
      pltpu.sync_copy(ob_v, out_h.at[pl.ds(r0, CHP)])

    dcnt.wait()
    for wsrc in range(NW):
      cnt = jnp.max(counts_v[wsrc])

      def abody(t, _, wsrc=wsrc):
        j = wsrc * NPW + t * 16
        pltpu.sync_copy(eidx_h.at[pl.ds(j, 16)], tgt16)
        pltpu.sync_copy(erow_h.at[pl.ds(j, 16)], rb_v)
        pltpu.async_copy(rb_v, out_h.at[tgt16], sem).wait()
        return 0

      lax.fori_loop(0, cnt // 16, abody, 0)

  return k(mp, mn, mc, mconf, counts, eidx, erow)


def kernel(map_points, map_normals, map_colors, map_confidence,
           frame_points, frame_normals, frame_colors, idx):
  counts, eidx, erow = _sc_find_fuse(
      map_points, map_normals, map_colors, map_confidence,
      frame_points, frame_normals, frame_colors, idx)
  return _sc_pack_apply(map_points, map_normals, map_colors, map_confidence,
                        counts, eidx, erow)


# speedup vs baseline: 2.2471x; 1.2383x over previous
import functools
import math

import jax
import jax.numpy as jnp
from jax import lax
from jax.experimental import pallas as pl
from jax.experimental.pallas import tpu as pltpu
from jax.experimental.pallas import tpu_sc as plsc

M = 1048576
N = 262144
DIST_TH2 = 0.05 * 0.05
DOT_TH = math.cos(20.0 * math.pi / 180.0)
DOT_TH2 = DOT_TH * DOT_TH
SIGMA = 0.6
INV_2SIG2 = 1.0 / (2.0 * SIGMA * SIGMA)

NC = 2
NS = 16
NW = NC * NS
NPW = N // NW
CH = 2048
NGC = CH // 16
RPW = M // NW
CHP = 2048
NGP = CHP // 16

_SC_PARAMS = pltpu.CompilerParams(
    needs_layout_passes=False, use_tc_tiling_on_sc=False)


def _col3(ref, rows3, c):
  return plsc.load_gather(ref, [rows3 + c])


def _sc_find_fuse(mp_f, mn_f, mc_f, mconf, fp_f, fn_f, fc_f, idx):
  mesh = plsc.VectorSubcoreMesh(core_axis_name="c", subcore_axis_name="s")

  @functools.partial(
      pl.kernel,
      out_type=(
          jax.ShapeDtypeStruct((NW, 16), jnp.int32),
          jax.ShapeDtypeStruct((N,), jnp.int32),
          jax.ShapeDtypeStruct((N, 10), jnp.float32),
      ),
      mesh=mesh,
      compiler_params=_SC_PARAMS,
      scratch_types=[
          pltpu.VMEM((CH,), jnp.int32),
          pltpu.VMEM((CH,), jnp.int32),
          pltpu.VMEM((CH,), jnp.int32),
          pltpu.VMEM((CH,), jnp.int32),
          pltpu.VMEM((3 * CH,), jnp.float32),
          pltpu.VMEM((3 * CH,), jnp.float32),
          pltpu.VMEM((CH,), jnp.float32),
          pltpu.VMEM((CH,), jnp.float32),
          pltpu.VMEM((CH,), jnp.float32),
          pltpu.VMEM((CH,), jnp.float32),
          pltpu.VMEM((CH,), jnp.float32),
          pltpu.VMEM((CH,), jnp.float32),
          pltpu.VMEM((NPW + 16,), jnp.int32),
          pltpu.VMEM((NPW + 16,), jnp.int32),
          pltpu.VMEM((16,), jnp.int32),
          pltpu.VMEM((16,), jnp.int32),
          pltpu.VMEM((16,), jnp.int32),
          pltpu.VMEM((16,), jnp.int32),
          pltpu.VMEM((16,), jnp.int32),
          pltpu.VMEM((16,), jnp.float32),
          pltpu.VMEM((16,), jnp.float32),
          pltpu.VMEM((16,), jnp.float32),
          pltpu.VMEM((16,), jnp.float32),
          pltpu.VMEM((16, 10), jnp.float32),
          pltpu.SemaphoreType.DMA,
      ],
  )
  def k(mp_h, mn_h, mc_h, mconf_h, fp_h, fn_h, fc_h, idx_h,
        counts_h, eidx_h, erow_h,
        idx_v, i3x, i3y, i3z, fp_v, fn_v,
        mpx_v, mpy_v, mpz_v, mnx_v, mny_v, mnz_v,
        gi_list, idx_list, cnt_v, ia_s, ib_s, ic_s, idx16,
        ax_s, ay_s, az_s, conf16, row16, sem):
    wid = lax.axis_index("s") * NC + lax.axis_index("c")
    base = wid * NPW
    iot = lax.iota(jnp.int32, 16)

    kcount = jnp.int32(0)
    for half in range(NPW // CH):
      cbase = base + half * CH

      d1 = pltpu.async_copy(idx_h.at[pl.ds(cbase, CH)], idx_v, sem)
      d2 = pltpu.async_copy(fp_h.at[pl.ds(3 * cbase, 3 * CH)], fp_v, sem)
      d3 = pltpu.async_copy(fn_h.at[pl.ds(3 * cbase, 3 * CH)], fn_v, sem)
      d1.wait()

      def mkidx(g, _):
        sl = pl.ds(g * 16, 16)
        i3 = idx_v[sl] * 3
        i3x[sl] = i3
        i3y[sl] = i3 + 1
        i3z[sl] = i3 + 2
        return 0

      lax.fori_loop(0, NGC, mkidx, 0)

      g1 = pltpu.async_copy(mp_h.at[i3x], mpx_v, sem)
      g2 = pltpu.async_copy(mp_h.at[i3y], mpy_v, sem)
      g3 = pltpu.async_copy(mp_h.at[i3z], mpz_v, sem)
      g4 = pltpu.async_copy(mn_h.at[i3x], mnx_v, sem)
      g5 = pltpu.async_copy(mn_h.at[i3y], mny_v, sem)
      g6 = pltpu.async_copy(mn_h.at[i3z], mnz_v, sem)
      d2.wait()
      d3.wait()
      g1.wait()
      g2.wait()
      g3.wait()
      g4.wait()
      g5.wait()
      g6.wait()

      def group(g, ptr, cbase=cbase):
        sl = pl.ds(g * 16, 16)
        rows3 = (g * 16 + iot) * 3
        fpx = _col3(fp_v, rows3, 0)
        fpy = _col3(fp_v, rows3, 1)
        fpz = _col3(fp_v, rows3, 2)
        dx = fpx - mpx_v[sl]
        dy = fpy - mpy_v[sl]
        dz = fpz - mpz_v[sl]
        d2_ = dx * dx + dy * dy + dz * dz
        fnx = _col3(fn_v, rows3, 0)
        fny = _col3(fn_v, rows3, 1)
        fnz = _col3(fn_v, rows3, 2)
        mnx = mnx_v[sl]
        mny = mny_v[sl]
        mnz = mnz_v[sl]
        dot = mnx * fnx + mny * fny + mnz * fnz
        nm2 = mnx * mnx + mny * mny + mnz * mnz
        nf2 = fnx * fnx + fny * fny + fnz * fnz
        valid = ((d2_ < DIST_TH2) & (dot > 0.0)
                 & (dot * dot > DOT_TH2 * nm2 * nf2))
        cnt = jnp.max(plsc.all_reduce_population_count(valid))

        @pl.when(cnt > 0)
        def _():
          gi_vals = cbase + g * 16 + iot
          idx16v = idx_v[sl]
          plsc.store_compressed(gi_list.at[pl.ds(ptr, 16)], gi_vals,
                                mask=valid)
          plsc.store_compressed(idx_list.at[pl.ds(ptr, 16)], idx16v,
                                mask=valid)

        return ptr + cnt

      kcount = lax.fori_loop(0, NGC, group, kcount)

    nfix = (kcount + 15) // 16
    cnt_v[...] = jnp.full((16,), nfix * 16, jnp.int32)
    pltpu.sync_copy(cnt_v, counts_h.at[wid])

    zeros16 = jnp.zeros((16,), jnp.int32)
    gi_list[pl.ds(kcount, 16)] = zeros16
    idx_list[pl.ds(kcount, 16)] = zeros16

    def gather3(src_h, base3):
      ia_s[...] = base3
      ib_s[...] = base3 + 1
      ic_s[...] = base3 + 2
      h1 = pltpu.async_copy(src_h.at[ia_s], ax_s, sem)
      h2 = pltpu.async_copy(src_h.at[ib_s], ay_s, sem)
      h3 = pltpu.async_copy(src_h.at[ic_s], az_s, sem)
      h1.wait()
      h2.wait()
      h3.wait()
      return ax_s[...], ay_s[...], az_s[...]

    def fix(t, _):
      e = t * 16
      gi3 = gi_list[pl.ds(e, 16)] * 3
      idx16v = idx_list[pl.ds(e, 16)]
      idx16[...] = idx16v
      i3 = idx16v * 3

      fpx, fpy, fpz = gather3(fp_h, gi3)
      r2 = fpx * fpx + fpy * fpy + fpz * fpz
      dc2 = (fpx * fpx + fpy * fpy) / (r2 + 1e-30)
      alpha = jnp.exp(-dc2 * INV_2SIG2)

      ia_s[...] = idx16v
      hc = pltpu.async_copy(mconf_h.at[ia_s], conf16, sem)
      hc.wait()
      w = conf16[...]

      real = (e + iot) < kcount
      inv_den = 1.0 / (w + alpha)
      wd = jnp.where(real, w * inv_den, 1.0)
      ad = jnp.where(real, alpha * inv_den, 0.0)

      c9 = jnp.full((16,), 9, jnp.int32)
      plsc.store_scatter(row16, [iot, c9], jnp.where(real, w + alpha, w))
      fps = (fpx, fpy, fpz)
      mps = gather3(mp_h, i3)
      for c in range(3):
        cc = jnp.full((16,), c, jnp.int32)
        plsc.store_scatter(row16, [iot, cc], wd * mps[c] + ad * fps[c])
      fns = gather3(fn_h, gi3)
      mns = gather3(mn_h, i3)
      for c in range(3):
        cc = jnp.full((16,), 3 + c, jnp.int32)
        plsc.store_scatter(row16, [iot, cc], wd * mns[c] + ad * fns[c])
      fcs = gather3(fc_h, gi3)
      mcs = gather3(mc_h, i3)
      for c in range(3):
        cc = jnp.full((16,), 6 + c, jnp.int32)
        plsc.store_scatter(row16, [iot, cc], wd * mcs[c] + ad * fcs[c])

      pltpu.sync_copy(row16, erow_h.at[pl.ds(base + e, 16)])
      pltpu.sync_copy(idx16, eidx_h.at[pl.ds(base + e, 16)])
      return 0

    lax.fori_loop(0, nfix, fix, 0)

  return k(mp_f, mn_f, mc_f, mconf, fp_f, fn_f, fc_f, idx)


def _sc_pack_apply(mp_f, mn_f, mc_f, mconf, counts, eidx, erow):
  mesh = plsc.VectorSubcoreMesh(core_axis_name="c", subcore_axis_name="s")

  @functools.partial(
      pl.kernel,
      out_type=jax.ShapeDtypeStruct((M, 10), jnp.float32),
      mesh=mesh,
      compiler_params=_SC_PARAMS,
      scratch_types=[
          pltpu.VMEM((3 * CHP,), jnp.float32),
          pltpu.VMEM((3 * CHP,), jnp.float32),
          pltpu.VMEM((3 * CHP,), jnp.float32),
          pltpu.VMEM((CHP,), jnp.float32),
          pltpu.VMEM((CHP, 10), jnp.float32),
          pltpu.VMEM((NW, 16), jnp.int32),
          pltpu.VMEM((16,), jnp.int32),
          pltpu.VMEM((16, 10), jnp.float32),
          pltpu.SemaphoreType.DMA,
      ],
  )
  def k(mp_h, mn_h, mc_h, mconf_h, counts_h, eidx_h, erow_h, out_h,
        mp_pv, mn_pv, mc_pv, cf_pv, ob_v, counts_v, tgt16, rb_v, sem):
    wid = lax.axis_index("s") * NC + lax.axis_index("c")
    rbase = wid * RPW
    iot = lax.iota(jnp.int32, 16)
    dcnt = pltpu.async_copy(counts_h, counts_v, sem)

    for chunk in range(RPW // CHP):
      r0 = rbase + chunk * CHP
      d1 = pltpu.async_copy(mp_h.at[pl.ds(3 * r0, 3 * CHP)], mp_pv, sem)
      d2 = pltpu.async_copy(mn_h.at[pl.ds(3 * r0, 3 * CHP)], mn_pv, sem)
      d3 = pltpu.async_copy(mc_h.at[pl.ds(3 * r0, 3 * CHP)], mc_pv, sem)
      d4 = pltpu.async_copy(mconf_h.at[pl.ds(r0, CHP)], cf_pv, sem)
      d1.wait()
      d2.wait()
      d3.wait()
      d4.wait()

      def pgroup(g, _):
        rows = g * 16 + iot
        rows3 = rows * 3
        for off, src in ((0, mp_pv), (3, mn_pv), (6, mc_pv)):
          for c in range(3):
            cc = jnp.full((16,), off + c, jnp.int32)
            plsc.store_scatter(ob_v, [rows, cc], _col3(src, rows3, c))
        c9 = jnp.full((16,), 9, jnp.int32)
        plsc.store_scatter(ob_v, [rows, c9], cf_pv[pl.ds(g * 16, 16)])
        return 0

      lax.fori_loop(0, NGP, pgroup, 0)
      pltpu.sync_copy(ob_v, out_h.at[pl.ds(r0, CHP)])

    dcnt.wait()
    for wsrc in range(NW):
      cnt = jnp.max(counts_v[wsrc])

      def abody(t, _, wsrc=wsrc):
        j = wsrc * NPW + t * 16
        pltpu.sync_copy(eidx_h.at[pl.ds(j, 16)], tgt16)
        pltpu.sync_copy(erow_h.at[pl.ds(j, 16)], rb_v)
        pltpu.async_copy(rb_v, out_h.at[tgt16], sem).wait()
        return 0

      lax.fori_loop(0, cnt // 16, abody, 0)

  return k(mp_f, mn_f, mc_f, mconf, counts, eidx, erow)


def kernel(map_points, map_normals, map_colors, map_confidence,
           frame_points, frame_normals, frame_colors, idx):
  mp_f = map_points.reshape(3 * M)
  mn_f = map_normals.reshape(3 * M)
  mc_f = map_colors.reshape(3 * M)
  fp_f = frame_points.reshape(3 * N)
  fn_f = frame_normals.reshape(3 * N)
  fc_f = frame_colors.reshape(3 * N)
  counts, eidx, erow = _sc_find_fuse(
      mp_f, mn_f, mc_f, map_confidence, fp_f, fn_f, fc_f, idx)
  return _sc_pack_apply(mp_f, mn_f, mc_f, map_confidence, counts, eidx, erow)
